# baseline (speedup 1.0000x reference)
import functools

import jax
import jax.numpy as jnp
from jax import lax
from jax.experimental import pallas as pl
from jax.experimental.pallas import tpu as pltpu
from jax.experimental.pallas import tpu_sc as plsc

B, L, D = 2, 2048, 768
H = 12
DH = D // H
GROUPS = 128
NPG = 16
SCALE = float(DH) ** -0.5

_DN = (((1,), (1,)), ((), ()))
_HI = jax.lax.Precision.HIGHEST

NW = 32
ROWS = H * GROUPS
RPW = ROWS // NW
BLK = 8
NBLK = RPW // BLK
IDXB = BLK * NPG
H2 = H // 2


def _dot_serial256(a, b):
    kdim = a.shape[1]
    acc = None
    for lo in range(0, kdim, 256):
        c = jax.lax.dot_general(a[:, lo:lo + 256], b[:, lo:lo + 256], _DN,
                                preferred_element_type=jnp.float32)
        acc = c if acc is None else acc + c
    return acc


def _select_body(x_ref, tt_ref, wq_ref, bq_ref, wk_ref, bk_ref, wv_ref,
                 bv_ref, v_ref, idx_ref):
    x = x_ref[0]

    k = _dot_serial256(x, wk_ref[...]) + bk_ref[0]
    v = _dot_serial256(x, wv_ref[...]) + bv_ref[0]
    q = _dot_serial256(tt_ref[...], wq_ref[...]) + bq_ref[0]
    v_ref[0, 0, :, :] = v

    s = jax.lax.dot_general(q, k, _DN,
                            preferred_element_type=jnp.float32) * SCALE

    m0 = jnp.max(s, axis=1, keepdims=True)
    u = jnp.exp(s - m0)
    s = u / jnp.sum(u, axis=1, keepdims=True)

    colidx = jax.lax.broadcasted_iota(jnp.int32, (GROUPS, L), 1)
    picks = []
    for t in range(NPG):
        m = jnp.max(s, axis=1, keepdims=True)
        cand = jnp.where(s == m, colidx, L)
        first = jnp.min(cand, axis=1, keepdims=True)
        picks.append(first)
        s = jnp.where(cand == first, -jnp.inf, s)
    idx_ref[0, 0] = jnp.concatenate(picks, axis=1)


def _sc_gather_combine(v2, idx3, gw_rep, gb_rep):
    mesh = plsc.VectorSubcoreMesh(core_axis_name="c", subcore_axis_name="s")

    @functools.partial(
        pl.kernel, mesh=mesh,
        out_type=jax.ShapeDtypeStruct((ROWS, 128), jnp.float32),
        scratch_types=[
            pltpu.VMEM((NBLK, IDXB), jnp.int32),
            pltpu.VMEM((IDXB, 128), jnp.float32),
            pltpu.VMEM((IDXB, 128), jnp.float32),
            pltpu.VMEM((GROUPS, NPG * 16), jnp.float32),
            pltpu.VMEM((GROUPS, 128), jnp.float32),
            pltpu.VMEM((RPW, 128), jnp.float32),
            pltpu.SemaphoreType.DMA,
            pltpu.SemaphoreType.DMA,
            pltpu.SemaphoreType.DMA,
        ],
    )
    def kern(v_hbm, idx_hbm, gw_hbm, gb_hbm, xo_hbm,
             idx_v, rows_v, rows_w, gw_v, gb_v, xo_v, sem, sem2, sem3):
        wid = lax.axis_index("s") * 2 + lax.axis_index("c")
        pltpu.sync_copy(idx_hbm.at[wid], idx_v)
        pltpu.sync_copy(gw_hbm, gw_v)
        pltpu.sync_copy(gb_hbm, gb_v)
        row0 = wid * RPW

        bufs = (rows_v, rows_w)
        sems = (sem, sem2)

        def combine(blk, buf):
            r0 = row0 + blk * BLK
            g0 = jnp.bitwise_and(r0, GROUPS - 1)
            off = jnp.bitwise_and(lax.shift_right_logical(r0, 7), 1) * DH
            for j in range(BLK):
                g = g0 + j
                gwt = [gw_v[g, pl.ds(t * 16, 16)] for t in range(NPG)]
                for c in range(DH // 16):
                    sl = pl.ds(off + c * 16, 16)
                    acc = gb_v[g, pl.ds(c * 16, 16)]
                    for t in range(NPG):
                        acc = acc + buf[j * NPG + t, sl] * gwt[t]
                    xo_v[blk * BLK + j, pl.ds(c * 16, 16)] = acc

        pltpu.async_copy(v_hbm.at[idx_v.at[0]], bufs[0], sems[0])

        @pl.loop(0, NBLK, step=2)
        def _(blk):
            pltpu.make_async_copy(v_hbm.at[idx_v.at[blk]],
                                  bufs[0], sems[0]).wait()
            pltpu.async_copy(v_hbm.at[idx_v.at[blk + 1]], bufs[1], sems[1])
            combine(blk, bufs[0])

            @pl.when(blk + 2 < NBLK)
            def _():
                pltpu.async_copy(v_hbm.at[idx_v.at[blk + 2]],
                                 bufs[0], sems[0])

            pltpu.make_async_copy(v_hbm.at[idx_v.at[blk + 1]],
                                  bufs[1], sems[1]).wait()
            combine(blk + 1, bufs[1])

        pltpu.async_copy(xo_v, xo_hbm.at[pl.ds(row0, RPW)], sem3).wait()

    return kern(v2, idx3, gw_rep, gb_rep)


def _outproj_body(xo_ref, wo_ref, bo_ref, out_ref):
    h = pl.program_id(1)
    contrib = jnp.dot(xo_ref[0, 0], wo_ref[0],
                      preferred_element_type=jnp.float32, precision=_HI)

    @pl.when(h == 0)
    def _():
        out_ref[0] = contrib + bo_ref[...]

    @pl.when(h != 0)
    def _():
        out_ref[0] = out_ref[0] + contrib


@jax.jit
def kernel(x, target_token, wq, bq, wk, bk, wv, bv, gather_w, gather_b,
           wo, bo):
    bq2 = bq.reshape(H, 1, DH)
    bk2 = bk.reshape(H, 1, DH)
    bv2 = bv.reshape(H, 1, DH)
    bo2 = bo.reshape(1, D)
    wo_t = wo.T.reshape(H, DH, D)

    gw_rep = jnp.broadcast_to(gather_w[:, :, None],
                              (GROUPS, NPG, 16)).reshape(GROUPS, NPG * 16)
    gb_rep = jnp.broadcast_to(gather_b[:, None], (GROUPS, 128))
    gw_rep = gw_rep + 0.0
    gb_rep = gb_rep + 0.0
    pair = jnp.arange(ROWS, dtype=jnp.int32) // (2 * GROUPS)

    xo_list = []
    for b in range(B):
        xb = lax.slice_in_dim(x, b, b + 1, axis=0)
        v_all, idx_all = pl.pallas_call(
            _select_body,
            grid=(1, H),
            in_specs=[
                pl.BlockSpec((1, L, D), lambda bb, h: (bb, 0, 0)),
                pl.BlockSpec((GROUPS, D), lambda bb, h: (0, 0)),
                pl.BlockSpec((DH, D), lambda bb, h: (h, 0)),
                pl.BlockSpec((1, 1, DH), lambda bb, h: (h, 0, 0)),
                pl.BlockSpec((DH, D), lambda bb, h: (h, 0)),
                pl.BlockSpec((1, 1, DH), lambda bb, h: (h, 0, 0)),
                pl.BlockSpec((DH, D), lambda bb, h: (h, 0)),
                pl.BlockSpec((1, 1, DH), lambda bb, h: (h, 0, 0)),
            ],
            out_specs=[
                pl.BlockSpec((1, 1, L, DH), lambda bb, h: (bb, h, 0, 0)),
                pl.BlockSpec((1, 1, GROUPS, NPG), lambda bb, h: (bb, h, 0, 0)),
            ],
            out_shape=[
                jax.ShapeDtypeStruct((1, H, L, DH), jnp.float32),
                jax.ShapeDtypeStruct((1, H, GROUPS, NPG), jnp.int32),
            ],
        )(xb, target_token, wq, bq2, wk, bk2, wv, bv2)

        v2 = v_all.reshape(H2, 2, L, DH).transpose(0, 2, 1, 3)
        v2 = v2.reshape(H2 * L, 2 * DH)
        idx_flat = idx_all.reshape(ROWS, NPG) + (pair * L)[:, None]
        idx3 = idx_flat.reshape(NW, NBLK, IDXB)
        xo = _sc_gather_combine(v2, idx3, gw_rep, gb_rep)
        xo_list.append(xo[:, :DH].reshape(1, H, GROUPS, DH))

    xo4 = jnp.concatenate(xo_list, axis=0)
    out = pl.pallas_call(
        _outproj_body,
        grid=(B, H),
        in_specs=[
            pl.BlockSpec((1, 1, GROUPS, DH), lambda b, h: (b, h, 0, 0)),
            pl.BlockSpec((1, DH, D), lambda b, h: (h, 0, 0)),
            pl.BlockSpec((1, D), lambda b, h: (0, 0)),
        ],
        out_specs=pl.BlockSpec((1, GROUPS, D), lambda b, h: (b, 0, 0)),
        out_shape=jax.ShapeDtypeStruct((B, GROUPS, D), jnp.float32),
    )(xo4, wo_t, bo2)
    return out


# speedup vs baseline: 1.1186x; 1.1186x over previous
import functools

import jax
import jax.numpy as jnp
from jax import lax
from jax.experimental import pallas as pl
from jax.experimental.pallas import tpu as pltpu
from jax.experimental.pallas import tpu_sc as plsc

B, L, D = 2, 2048, 768
H = 12
DH = D // H
GROUPS = 128
NPG = 16
SCALE = float(DH) ** -0.5

_DN = (((1,), (1,)), ((), ()))
_HI = jax.lax.Precision.HIGHEST

NW = 32
ROWS = B * H * GROUPS
RPW = ROWS // NW
BLK = 8
NBLK = RPW // BLK
IDXB = BLK * NPG
H2 = H // 2


def _dot_serial256(a, b):
    kdim = a.shape[1]
    acc = None
    for lo in range(0, kdim, 256):
        c = jax.lax.dot_general(a[:, lo:lo + 256], b[:, lo:lo + 256], _DN,
                                preferred_element_type=jnp.float32)
        acc = c if acc is None else acc + c
    return acc


def _select_body(x_ref, tt_ref, wq_ref, bq_ref, wk_ref, bk_ref, wv_ref,
                 bv_ref, v_ref, idx_ref):
    x = x_ref[0]

    k = _dot_serial256(x, wk_ref[...]) + bk_ref[0]
    v = _dot_serial256(x, wv_ref[...]) + bv_ref[0]
    q = _dot_serial256(tt_ref[...], wq_ref[...]) + bq_ref[0]
    v_ref[0, 0, :, :] = v

    s = jax.lax.dot_general(q, k, _DN,
                            preferred_element_type=jnp.float32) * SCALE

    m0 = jnp.max(s, axis=1, keepdims=True)
    u = jnp.exp(s - m0)
    s = u / jnp.sum(u, axis=1, keepdims=True)

    colidx = jax.lax.broadcasted_iota(jnp.int32, (GROUPS, L), 1)
    picks = []
    for t in range(NPG):
        m = jnp.max(s, axis=1, keepdims=True)
        cand = jnp.where(s == m, colidx, L)
        first = jnp.min(cand, axis=1, keepdims=True)
        picks.append(first)
        s = jnp.where(cand == first, -jnp.inf, s)
    idx_ref[0, 0] = jnp.concatenate(picks, axis=1)


def _sc_gather_combine(v2, idx3, gw_rep, gb_rep):
    mesh = plsc.VectorSubcoreMesh(core_axis_name="c", subcore_axis_name="s")

    @functools.partial(
        pl.kernel, mesh=mesh,
        out_type=jax.ShapeDtypeStruct((ROWS, 128), jnp.float32),
        scratch_types=[
            pltpu.VMEM((NBLK, IDXB), jnp.int32),
            pltpu.VMEM((IDXB, 128), jnp.float32),
            pltpu.VMEM((IDXB, 128), jnp.float32),
            pltpu.VMEM((GROUPS, NPG * 16), jnp.float32),
            pltpu.VMEM((GROUPS, 128), jnp.float32),
            pltpu.VMEM((RPW, 128), jnp.float32),
            pltpu.SemaphoreType.DMA,
            pltpu.SemaphoreType.DMA,
            pltpu.SemaphoreType.DMA,
        ],
    )
    def kern(v_hbm, idx_hbm, gw_hbm, gb_hbm, xo_hbm,
             idx_v, rows_v, rows_w, gw_v, gb_v, xo_v, sem, sem2, sem3):
        wid = lax.axis_index("s") * 2 + lax.axis_index("c")
        pltpu.sync_copy(idx_hbm.at[wid], idx_v)
        pltpu.sync_copy(gw_hbm, gw_v)
        pltpu.sync_copy(gb_hbm, gb_v)
        row0 = wid * RPW

        bufs = (rows_v, rows_w)
        sems = (sem, sem2)

        def combine(blk, buf):
            r0 = row0 + blk * BLK
            g0 = jnp.bitwise_and(r0, GROUPS - 1)
            off = jnp.bitwise_and(lax.shift_right_logical(r0, 7), 1) * DH
            for j in range(BLK):
                g = g0 + j
                gwt = [gw_v[g, pl.ds(t * 16, 16)] for t in range(NPG)]
                for c in range(DH // 16):
                    sl = pl.ds(off + c * 16, 16)
                    acc = gb_v[g, pl.ds(c * 16, 16)]
                    for t in range(NPG):
                        acc = acc + buf[j * NPG + t, sl] * gwt[t]
                    xo_v[blk * BLK + j, pl.ds(c * 16, 16)] = acc

        pltpu.async_copy(v_hbm.at[idx_v.at[0]], bufs[0], sems[0])

        @pl.loop(0, NBLK, step=2)
        def _(blk):
            pltpu.make_async_copy(v_hbm.at[idx_v.at[blk]],
                                  bufs[0], sems[0]).wait()
            pltpu.async_copy(v_hbm.at[idx_v.at[blk + 1]], bufs[1], sems[1])
            combine(blk, bufs[0])

            @pl.when(blk + 2 < NBLK)
            def _():
                pltpu.async_copy(v_hbm.at[idx_v.at[blk + 2]],
                                 bufs[0], sems[0])

            pltpu.make_async_copy(v_hbm.at[idx_v.at[blk + 1]],
                                  bufs[1], sems[1]).wait()
            combine(blk + 1, bufs[1])

        pltpu.async_copy(xo_v, xo_hbm.at[pl.ds(row0, RPW)], sem3).wait()

    return kern(v2, idx3, gw_rep, gb_rep)


def _outproj_body(xo_ref, wo_ref, bo_ref, out_ref):
    h = pl.program_id(1)
    contrib = jnp.dot(xo_ref[0, 0], wo_ref[0],
                      preferred_element_type=jnp.float32, precision=_HI)

    @pl.when(h == 0)
    def _():
        out_ref[0] = contrib + bo_ref[...]

    @pl.when(h != 0)
    def _():
        out_ref[0] = out_ref[0] + contrib


@jax.jit
def kernel(x, target_token, wq, bq, wk, bk, wv, bv, gather_w, gather_b,
           wo, bo):
    bq2 = bq.reshape(H, 1, DH)
    bk2 = bk.reshape(H, 1, DH)
    bv2 = bv.reshape(H, 1, DH)
    bo2 = bo.reshape(1, D)
    wo_t = wo.T.reshape(H, DH, D)

    v_all, idx_all = pl.pallas_call(
        _select_body,
        grid=(B, H),
        in_specs=[
            pl.BlockSpec((1, L, D), lambda b, h: (b, 0, 0)),
            pl.BlockSpec((GROUPS, D), lambda b, h: (0, 0)),
            pl.BlockSpec((DH, D), lambda b, h: (h, 0)),
            pl.BlockSpec((1, 1, DH), lambda b, h: (h, 0, 0)),
            pl.BlockSpec((DH, D), lambda b, h: (h, 0)),
            pl.BlockSpec((1, 1, DH), lambda b, h: (h, 0, 0)),
            pl.BlockSpec((DH, D), lambda b, h: (h, 0)),
            pl.BlockSpec((1, 1, DH), lambda b, h: (h, 0, 0)),
        ],
        out_specs=[
            pl.BlockSpec((1, 1, L, DH), lambda b, h: (b, h, 0, 0)),
            pl.BlockSpec((1, 1, GROUPS, NPG), lambda b, h: (b, h, 0, 0)),
        ],
        out_shape=[
            jax.ShapeDtypeStruct((B, H, L, DH), jnp.float32),
            jax.ShapeDtypeStruct((B, H, GROUPS, NPG), jnp.int32),
        ],
    )(x, target_token, wq, bq2, wk, bk2, wv, bv2)

    v2 = v_all.reshape(B, H2, 2, L, DH).transpose(0, 1, 3, 2, 4)
    v2 = v2.reshape(B * H2 * L, 2 * DH)
    pair = jnp.arange(ROWS, dtype=jnp.int32) // (2 * GROUPS)
    idx_flat = idx_all.reshape(ROWS, NPG) + (pair * L)[:, None]
    idx3 = idx_flat.reshape(NW, NBLK, IDXB)
    gw_rep = jnp.broadcast_to(gather_w[:, :, None],
                              (GROUPS, NPG, 16)).reshape(GROUPS, NPG * 16)
    gb_rep = jnp.broadcast_to(gather_b[:, None], (GROUPS, 128))

    xo = _sc_gather_combine(v2, idx3, gw_rep + 0.0, gb_rep + 0.0)
    xo4 = xo[:, :DH].reshape(B, H, GROUPS, DH)

    out = pl.pallas_call(
        _outproj_body,
        grid=(B, H),
        in_specs=[
            pl.BlockSpec((1, 1, GROUPS, DH), lambda b, h: (b, h, 0, 0)),
            pl.BlockSpec((1, DH, D), lambda b, h: (h, 0, 0)),
            pl.BlockSpec((1, D), lambda b, h: (0, 0)),
        ],
        out_specs=pl.BlockSpec((1, GROUPS, D), lambda b, h: (b, 0, 0)),
        out_shape=jax.ShapeDtypeStruct((B, GROUPS, D), jnp.float32),
    )(xo4, wo_t, bo2)
    return out
